# baseline (speedup 1.0000x reference)
import functools

import jax
import jax.numpy as jnp
from jax import lax
from jax.experimental import pallas as pl
from jax.experimental.pallas import tpu as pltpu
from jax.experimental.pallas import tpu_sc as plsc

N_EMBD = 64
LANES = 16
NC = 2
NS = 16
NW = NC * NS

CH = 256
SUB = 128
NSUB = CH // SUB
OP = CH // 2


def _fire_idx(x2_hbm, idx3, isem, wblk, g):
    pltpu.async_copy(x2_hbm.at[wblk + g], idx3.at[lax.rem(g, 3)], isem)


def _wait_idx(x2_hbm, idx3, isem):
    pltpu.make_async_copy(x2_hbm.at[0], idx3.at[0], isem).wait()


def _compute_jj(idx3, jj2, g, jb):
    g3 = lax.rem(g, 3)
    for v in range(CH // LANES):
        vec = idx3[g3, pl.ds(v * LANES, LANES)]
        jj2[jb, pl.ds(v * LANES, LANES)] = lax.shift_right_logical(vec, 1)


def _fire_gather(tpair_hbm, jj2, jb, gbuf, gsem):
    for j in range(NSUB):
        pltpu.async_copy(
            tpair_hbm.at[jj2.at[jb, pl.ds(j * SUB, SUB)]],
            gbuf.at[pl.ds(j * SUB, SUB)],
            gsem,
        )


def _wait_gather(tpair_hbm, jj2, gbuf, gsem):
    for j in range(NSUB):
        pltpu.make_async_copy(
            tpair_hbm.at[jj2.at[0, pl.ds(j * SUB, SUB)]],
            gbuf.at[pl.ds(j * SUB, SUB)],
            gsem,
        ).wait()


def _fire_out(obuf, out_hbm, osem, wpair, g):
    pltpu.async_copy(obuf, out_hbm.at[pl.ds(wpair + g * OP, OP)], osem)


def _wait_out(obuf, out_hbm, osem):
    pltpu.make_async_copy(obuf, out_hbm.at[pl.ds(0, OP)], osem).wait()


def _normalize(idx3, gbuf, obuf, g):
    g3 = lax.rem(g, 3)
    ii = lax.iota(jnp.int32, LANES)

    def group_body(m16, carry):
        rb = m16 * LANES
        iv = idx3[g3, pl.ds(rb, LANES)]
        hb = (iv & 1) * N_EMBD
        ob_base = m16 * (LANES // 2)
        for k in range(LANES):
            r = rb + k
            half = k % 2
            m = ob_base + k // 2
            hbase = hb[k]
            va = gbuf[r, pl.ds(hbase, LANES)]
            vb = gbuf[r, pl.ds(hbase + LANES, LANES)]
            vc = gbuf[r, pl.ds(hbase + 2 * LANES, LANES)]
            vd = gbuf[r, pl.ds(hbase + 3 * LANES, LANES)]
            s = va * va + vb * vb + vc * vc + vd * vd
            for step in (8, 4, 2, 1):
                s = s + s.at[ii ^ step].get(mode="promise_in_bounds")
            i = lax.bitcast_convert_type(s, jnp.int32)
            i = jnp.full((LANES,), 0x5F3759DF, jnp.int32) - lax.shift_right_logical(i, 1)
            y = lax.bitcast_convert_type(i, jnp.float32)
            h = 0.5 * s
            y = y * (1.5 - h * y * y)
            y = y * (1.5 - h * y * y)
            y = y * (1.5 - h * y * y)
            ob = half * N_EMBD
            obuf[m, pl.ds(ob, LANES)] = va * y
            obuf[m, pl.ds(ob + LANES, LANES)] = vb * y
            obuf[m, pl.ds(ob + 2 * LANES, LANES)] = vc * y
            obuf[m, pl.ds(ob + 3 * LANES, LANES)] = vd * y
        return carry

    lax.fori_loop(0, CH // LANES, group_body, 0)


def _body(x2_hbm, tpair_hbm, out_hbm, idx3, jj2, gbuf0, gbuf1,
          obuf0, obuf1, isem, gsem0, gsem1, osem0, osem1):
    wid = lax.axis_index("s") * NC + lax.axis_index("c")
    nch = x2_hbm.shape[0] // NW
    wblk = wid * nch
    wpair = wid * nch * OP

    gbuf = (gbuf0, gbuf1)
    obuf = (obuf0, obuf1)
    gsem = (gsem0, gsem1)
    osem = (osem0, osem1)

    def step(g, b, first):
        nb = 1 - b
        _wait_idx(x2_hbm, idx3, isem)
        _compute_jj(idx3, jj2, g + 1, nb)
        _fire_gather(tpair_hbm, jj2, nb, gbuf[nb], gsem[nb])

        @pl.when(g + 2 < nch)
        def _():
            _fire_idx(x2_hbm, idx3, isem, wblk, g + 2)

        _wait_gather(tpair_hbm, jj2, gbuf[b], gsem[b])
        if not first:
            _wait_out(obuf[b], out_hbm, osem[b])
        _normalize(idx3, gbuf[b], obuf[b], g)
        _fire_out(obuf[b], out_hbm, osem[b], wpair, g)

    pltpu.sync_copy(x2_hbm.at[wblk], idx3.at[0])
    _compute_jj(idx3, jj2, 0, 0)
    _fire_gather(tpair_hbm, jj2, 0, gbuf[0], gsem[0])
    _fire_idx(x2_hbm, idx3, isem, wblk, 1)

    step(0, 0, True)
    step(1, 1, True)
    step(2, 0, False)

    def pair(i, carry):
        step(3 + 2 * i, 1, False)
        step(4 + 2 * i, 0, False)
        return carry

    lax.fori_loop(0, (nch - 4) // 2, pair, 0)

    gl = nch - 1
    _wait_gather(tpair_hbm, jj2, gbuf[1], gsem[1])
    _wait_out(obuf[1], out_hbm, osem[1])
    _normalize(idx3, gbuf[1], obuf[1], gl)
    _fire_out(obuf[1], out_hbm, osem[1], wpair, gl)
    _wait_out(obuf[0], out_hbm, osem[0])
    _wait_out(obuf[1], out_hbm, osem[1])


def kernel(x, table):
    B = x.shape[0] * x.shape[1]
    V = table.shape[0]
    nch = B // (NW * CH)
    assert B % (NW * CH) == 0 and nch % 2 == 0 and nch >= 6 and V % 2 == 0
    x2 = jnp.reshape(x, (B // CH, CH)).astype(jnp.int32)
    tpair = jnp.reshape(table, (V // 2, 2 * N_EMBD))
    mesh = plsc.VectorSubcoreMesh(core_axis_name="c", subcore_axis_name="s")
    run = functools.partial(
        pl.kernel,
        out_type=jax.ShapeDtypeStruct((B // 2, 2 * N_EMBD), jnp.float32),
        mesh=mesh,
        scratch_types=[
            pltpu.VMEM((3, CH), jnp.int32),
            pltpu.VMEM((2, CH), jnp.int32),
            pltpu.VMEM((CH, 2 * N_EMBD), jnp.float32),
            pltpu.VMEM((CH, 2 * N_EMBD), jnp.float32),
            pltpu.VMEM((OP, 2 * N_EMBD), jnp.float32),
            pltpu.VMEM((OP, 2 * N_EMBD), jnp.float32),
            pltpu.SemaphoreType.DMA,
            pltpu.SemaphoreType.DMA,
            pltpu.SemaphoreType.DMA,
            pltpu.SemaphoreType.DMA,
            pltpu.SemaphoreType.DMA,
        ],
        compiler_params=pltpu.CompilerParams(use_tc_tiling_on_sc=False),
    )(_body)
    out = run(x2, tpair)
    return jnp.reshape(out, (x.shape[0], x.shape[1], N_EMBD))


# speedup vs baseline: 1.4681x; 1.4681x over previous
import functools

import jax
import jax.numpy as jnp
from jax import lax
from jax.experimental import pallas as pl
from jax.experimental.pallas import tpu as pltpu
from jax.experimental.pallas import tpu_sc as plsc

N_EMBD = 64
LANES = 16
NC = 2
NS = 16
NW = NC * NS

CH = 512
SUB = 128
NSUB = CH // SUB
UNROLL = 4


def _fire_gather(table_hbm, idx_all, rows, sem, g):
    for j in range(NSUB):
        pltpu.async_copy(
            table_hbm.at[idx_all.at[pl.ds(g * CH + j * SUB, SUB)]],
            rows.at[pl.ds(j * SUB, SUB)],
            sem,
        )


def _wait_gather(table_hbm, idx_all, rows, sem):
    for j in range(NSUB):
        pltpu.make_async_copy(
            table_hbm.at[idx_all.at[pl.ds(j * SUB, SUB)]],
            rows.at[pl.ds(j * SUB, SUB)],
            sem,
        ).wait()


def _wait_out(rows, out_hbm, sem):
    pltpu.make_async_copy(rows, out_hbm.at[pl.ds(0, CH)], sem).wait()


def _compute(rows):
    ii = lax.iota(jnp.int32, LANES)

    def quad(r, rcarry):
        rb = r * UNROLL
        for k in range(UNROLL):
            row = rb + k
            va = rows[row, pl.ds(0, LANES)]
            vb = rows[row, pl.ds(LANES, LANES)]
            vc = rows[row, pl.ds(2 * LANES, LANES)]
            vd = rows[row, pl.ds(3 * LANES, LANES)]
            s = va * va + vb * vb + vc * vc + vd * vd
            for step in (8, 4, 2, 1):
                s = s + s.at[ii ^ step].get(mode="promise_in_bounds")
            i = lax.bitcast_convert_type(s, jnp.int32)
            i = jnp.full((LANES,), 0x5F3759DF, jnp.int32) - lax.shift_right_logical(i, 1)
            y = lax.bitcast_convert_type(i, jnp.float32)
            h = 0.5 * s
            y = y * (1.5 - h * y * y)
            y = y * (1.5 - h * y * y)
            y = y * (1.5 - h * y * y)
            rows[row, pl.ds(0, LANES)] = va * y
            rows[row, pl.ds(LANES, LANES)] = vb * y
            rows[row, pl.ds(2 * LANES, LANES)] = vc * y
            rows[row, pl.ds(3 * LANES, LANES)] = vd * y
        return rcarry

    lax.fori_loop(0, CH // UNROLL, quad, 0)


def _body(x_hbm, table_hbm, out_hbm, idx_all, rows0, rows1,
          gsem0, gsem1, osem0, osem1):
    wid = lax.axis_index("s") * NC + lax.axis_index("c")
    b_per_w = x_hbm.shape[0] // NW
    nch = b_per_w // CH
    base = wid * b_per_w

    rows = (rows0, rows1)
    gsem = (gsem0, gsem1)
    osem = (osem0, osem1)

    pltpu.sync_copy(x_hbm.at[pl.ds(base, b_per_w)], idx_all)

    _fire_gather(table_hbm, idx_all, rows0, gsem0, 0)
    _fire_gather(table_hbm, idx_all, rows1, gsem1, 1)
    _wait_gather(table_hbm, idx_all, rows0, gsem0)
    _compute(rows0)
    pltpu.async_copy(rows0, out_hbm.at[pl.ds(base, CH)], osem0)

    def pair(i, carry):
        for off in range(2):
            g = 1 + 2 * i + off
            b = (1 + off) % 2
            nb = 1 - b
            _wait_out(rows[nb], out_hbm, osem[nb])
            _fire_gather(table_hbm, idx_all, rows[nb], gsem[nb], g + 1)
            _wait_gather(table_hbm, idx_all, rows[b], gsem[b])
            _compute(rows[b])
            pltpu.async_copy(rows[b], out_hbm.at[pl.ds(base + g * CH, CH)], osem[b])
        return carry

    lax.fori_loop(0, (nch - 2) // 2, pair, 0)

    gl = nch - 1
    bl = gl % 2
    _wait_gather(table_hbm, idx_all, rows[bl], gsem[bl])
    _compute(rows[bl])
    pltpu.async_copy(rows[bl], out_hbm.at[pl.ds(base + gl * CH, CH)], osem[bl])
    _wait_out(rows[0], out_hbm, osem[0])
    _wait_out(rows[1], out_hbm, osem[1])


def kernel(x, table):
    B = x.shape[0] * x.shape[1]
    b_per_w = B // NW
    nch = b_per_w // CH
    assert B % NW == 0 and b_per_w % CH == 0 and nch % 2 == 0 and nch >= 4
    xf = jnp.reshape(x, (B,)).astype(jnp.int32)
    mesh = plsc.VectorSubcoreMesh(core_axis_name="c", subcore_axis_name="s")
    run = functools.partial(
        pl.kernel,
        out_type=jax.ShapeDtypeStruct((B, N_EMBD), jnp.float32),
        mesh=mesh,
        scratch_types=[
            pltpu.VMEM((b_per_w,), jnp.int32),
            pltpu.VMEM((CH, N_EMBD), jnp.float32),
            pltpu.VMEM((CH, N_EMBD), jnp.float32),
            pltpu.SemaphoreType.DMA,
            pltpu.SemaphoreType.DMA,
            pltpu.SemaphoreType.DMA,
            pltpu.SemaphoreType.DMA,
        ],
        compiler_params=pltpu.CompilerParams(use_tc_tiling_on_sc=False),
    )(_body)
    out = run(xf, table)
    return jnp.reshape(out, (x.shape[0], x.shape[1], N_EMBD))
